# baseline (speedup 1.0000x reference)
import jax
import jax.numpy as jnp
from jax import lax
from jax.experimental import pallas as pl
from jax.experimental.pallas import tpu as pltpu
from jax.experimental.pallas import tpu_sc as plsc

NC = 2
NS = 16
L = 16

V = 100_000
V_PAD = 102_400
F = 200_000
F_PAD = 204_800
NF_TILE = F_PAD // NS
CHUNK = 512
NCHUNK = NF_TILE // CHUNK
SUB = CHUNK // 128
NVB_TILE = V_PAD // NS
NVF_TILE = V // NS
PB = 1600


def _iota16():
    return lax.iota(jnp.int32, L)


def _full16(v):
    return jnp.full((L,), v, dtype=jnp.int32)


def _rsqrt(s):
    i = plsc.bitcast(s, jnp.int32)
    i = 0x5F3759DF - lax.shift_right_arithmetic(i, 1)
    y = plsc.bitcast(i, jnp.float32)
    h = 0.5 * s
    for _ in range(3):
        y = y * (1.5 - h * y * y)
    return y


def _sc_body(verts, faces_f, out, areas, table,
             vbuf, bbuf, fbuf, idxg, idxs0, idxs1, idxs2,
             g0, g1, g2, nrm, ar0, ar1, zbuf, acc, sem, zsem):
    c = lax.axis_index("c")
    s = lax.axis_index("s")
    tile_face0 = s * NF_TILE
    iota = _iota16()
    zero_f = jnp.zeros((L,), jnp.float32)
    zero_i = jnp.zeros((L,), jnp.int32)

    def zb(i, _):
        rows = 2 * i + lax.shift_right_logical(iota, 3)
        cols = lax.bitwise_and(iota, _full16(7))
        plsc.store_scatter(zbuf, [rows, cols], zero_f)
        return _
    lax.fori_loop(0, 32, zb, None)

    def zn(i, _):
        rows = i * L + iota
        plsc.store_scatter(nrm, [rows, _full16(3)], zero_f)
        plsc.store_scatter(nrm, [rows, _full16(7)], zero_f)
        return _
    lax.fori_loop(0, CHUNK // L, zn, None)

    zds = []
    for i in range(NVB_TILE // 64):
        zds.append(pltpu.async_copy(
            zbuf, acc.at[pl.ds(s * NVB_TILE + i * 64, 64)], zsem))

    vb = s * NVB_TILE
    for p in range(NVB_TILE // PB):
        for b in (0, 1):
            base = (2 * c + b) * (3 * V_PAD) + (vb + p * PB) * 3
            pltpu.sync_copy(verts.at[pl.ds(base, 3 * PB)], vbuf)

            def bld(i, _):
                rows = i * L + iota
                r3 = 3 * rows
                x = plsc.load_gather(vbuf, [r3])
                y = plsc.load_gather(vbuf, [r3 + 1])
                z = plsc.load_gather(vbuf, [r3 + 2])
                o = _full16(4 * b)
                plsc.store_scatter(bbuf, [rows, o], x)
                plsc.store_scatter(bbuf, [rows, o + 1], y)
                plsc.store_scatter(bbuf, [rows, o + 2], z)
                return _
            lax.fori_loop(0, PB // L, bld, None)

        pltpu.sync_copy(
            bbuf, table.at[pl.ds(c * V_PAD + vb + p * PB, PB)])
    for d in zds:
        d.wait()
    plsc.subcore_barrier()

    coff = c * V_PAD

    def chunk_body(j, _):
        fb = tile_face0 + j * CHUNK
        pltpu.sync_copy(faces_f.at[pl.ds(fb * 3, 3 * CHUNK)], fbuf)

        def ext(i, _):
            rows = i * L + iota
            r3 = 3 * rows
            v0 = plsc.load_gather(fbuf, [r3])
            v1 = plsc.load_gather(fbuf, [r3 + 1])
            v2 = plsc.load_gather(fbuf, [r3 + 2])
            u = i // (128 // L)
            e = (i % (128 // L)) * L + iota
            plsc.store_scatter(idxs0, [_full16(0) + u, e], v0)
            plsc.store_scatter(idxs1, [_full16(0) + u, e], v1)
            plsc.store_scatter(idxs2, [_full16(0) + u, e], v2)
            plsc.store_scatter(idxg, [_full16(0) + u, e], v0 + coff)
            plsc.store_scatter(idxg, [_full16(SUB) + u, e], v1 + coff)
            plsc.store_scatter(idxg, [_full16(2 * SUB) + u, e], v2 + coff)
            return _
        lax.fori_loop(0, CHUNK // L, ext, None)

        descs = []
        for u in range(SUB):
            descs.append(pltpu.async_copy(
                table.at[idxg.at[u]], g0.at[pl.ds(u * 128, 128)], sem))
            descs.append(pltpu.async_copy(
                table.at[idxg.at[SUB + u]], g1.at[pl.ds(u * 128, 128)], sem))
            descs.append(pltpu.async_copy(
                table.at[idxg.at[2 * SUB + u]], g2.at[pl.ds(u * 128, 128)], sem))
        for d in descs:
            d.wait()

        def step(i, _):
            rows = i * L + iota
            for b in (0, 1):
                o = 4 * b
                ax = plsc.load_gather(g0, [rows, _full16(o)])
                ay = plsc.load_gather(g0, [rows, _full16(o + 1)])
                az = plsc.load_gather(g0, [rows, _full16(o + 2)])
                bx = plsc.load_gather(g1, [rows, _full16(o)])
                by = plsc.load_gather(g1, [rows, _full16(o + 1)])
                bz = plsc.load_gather(g1, [rows, _full16(o + 2)])
                cx = plsc.load_gather(g2, [rows, _full16(o)])
                cy = plsc.load_gather(g2, [rows, _full16(o + 1)])
                cz = plsc.load_gather(g2, [rows, _full16(o + 2)])
                e1x, e1y, e1z = bx - ax, by - ay, bz - az
                e2x, e2y, e2z = cx - bx, cy - by, cz - bz
                nx = e1y * e2z - e1z * e2y
                ny = e1z * e2x - e1x * e2z
                nz = e1x * e2y - e1y * e2x
                plsc.store_scatter(nrm, [rows, _full16(o)], nx)
                plsc.store_scatter(nrm, [rows, _full16(o + 1)], ny)
                plsc.store_scatter(nrm, [rows, _full16(o + 2)], nz)
                sq = nx * nx + ny * ny + nz * nz
                area = 0.5 * sq * _rsqrt(sq)
                ar = ar0 if b == 0 else ar1
                ar[pl.ds(i * L, L)] = area
            return _
        lax.fori_loop(0, CHUNK // L, step, None)

        for u in range(SUB):
            sl = pl.ds(u * 128, 128)
            pltpu.sync_copy(nrm.at[sl], acc.at[idxs0.at[u]], add=True)
            pltpu.sync_copy(nrm.at[sl], acc.at[idxs1.at[u]], add=True)
            pltpu.sync_copy(nrm.at[sl], acc.at[idxs2.at[u]], add=True)

        for b in (0, 1):
            ar = ar0 if b == 0 else ar1

            @pl.when(fb + CHUNK <= F)
            def _():
                pltpu.sync_copy(ar, areas.at[2 * c + b, pl.ds(fb, CHUNK)])

            @pl.when(fb == (F // CHUNK) * CHUNK)
            def _():
                rem = F - (F // CHUNK) * CHUNK
                pltpu.sync_copy(ar.at[pl.ds(0, rem)],
                                areas.at[2 * c + b, pl.ds(fb, rem)])
        return _
    lax.fori_loop(0, NCHUNK, chunk_body, None)

    plsc.subcore_barrier()

    vf = s * NVF_TILE
    for q in range(4):
        nq = PB if q < 3 else NVF_TILE - 3 * PB
        pltpu.sync_copy(acc.at[pl.ds(vf + q * PB, nq)],
                        bbuf.at[pl.ds(0, nq)])

        def fstep(i, _):
            rows = i * L + iota
            for b in (0, 1):
                o = 4 * b
                x = plsc.load_gather(bbuf, [rows, _full16(o)])
                y = plsc.load_gather(bbuf, [rows, _full16(o + 1)])
                z = plsc.load_gather(bbuf, [rows, _full16(o + 2)])
                sq = x * x + y * y + z * z
                r = jnp.where(sq >= 1e-12, _rsqrt(sq), 1e6)
                plsc.store_scatter(bbuf, [rows, _full16(o)], x * r)
                plsc.store_scatter(bbuf, [rows, _full16(o + 1)], y * r)
                plsc.store_scatter(bbuf, [rows, _full16(o + 2)], z * r)
            return _
        lax.fori_loop(0, (nq + L - 1) // L, fstep, None)

        for b in (0, 1):
            pltpu.sync_copy(bbuf.at[pl.ds(0, nq), pl.ds(4 * b, 3)],
                            out.at[2 * c + b, pl.ds(vf + q * PB, nq), :])


@jax.jit
def kernel(vertices, faces):
    faces = jnp.squeeze(faces)
    verts_f = jnp.pad(vertices, ((0, 0), (0, V_PAD - V), (0, 0))).reshape(-1)
    faces_f = jnp.pad(faces, ((0, F_PAD - F), (0, 0))).reshape(-1)

    mesh = plsc.VectorSubcoreMesh(core_axis_name="c", subcore_axis_name="s")
    run = pl.kernel(
        _sc_body,
        out_type=(
            jax.ShapeDtypeStruct((4, V, 3), jnp.float32),
            jax.ShapeDtypeStruct((4, F), jnp.float32),
            jax.ShapeDtypeStruct((NC * V_PAD, 8), jnp.float32),
        ),
        mesh=mesh,
        compiler_params=pltpu.CompilerParams(
            use_tc_tiling_on_sc=False, needs_layout_passes=False),
        scratch_types=(
            pltpu.VMEM((3 * PB,), jnp.float32),
            pltpu.VMEM((PB, 8), jnp.float32),
            pltpu.VMEM((3 * CHUNK,), jnp.int32),
            pltpu.VMEM((3 * SUB, 128), jnp.int32),
            pltpu.VMEM((SUB, 128), jnp.int32),
            pltpu.VMEM((SUB, 128), jnp.int32),
            pltpu.VMEM((SUB, 128), jnp.int32),
            pltpu.VMEM((CHUNK, 8), jnp.float32),
            pltpu.VMEM((CHUNK, 8), jnp.float32),
            pltpu.VMEM((CHUNK, 8), jnp.float32),
            pltpu.VMEM((CHUNK, 8), jnp.float32),
            pltpu.VMEM((CHUNK,), jnp.float32),
            pltpu.VMEM((CHUNK,), jnp.float32),
            pltpu.VMEM((64, 8), jnp.float32),
            pltpu.VMEM_SHARED((V_PAD, 8), jnp.float32),
            pltpu.SemaphoreType.DMA,
            pltpu.SemaphoreType.DMA,
        ),
    )
    vectors, areas_out, _ = run(verts_f, faces_f)
    return (vectors, areas_out)
